# Initial kernel scaffold; baseline (speedup 1.0000x reference)
import jax
import jax.numpy as jnp
from jax.experimental import pallas as pl


def kernel(x, W, b):
    raise NotImplementedError("write your pallas kernel here")


# speedup vs baseline: 3.8293x; 3.8293x over previous
import functools

import jax
import jax.numpy as jnp
import numpy as np
from jax.experimental import pallas as pl
from jax.experimental.pallas import tpu as pltpu

_DEPTH = 8
_R = 2 ** _DEPTH
_NPAD = _R


def _route_matrix() -> np.ndarray:
    a = np.zeros((2 * _NPAD, _R), dtype=np.float32)
    for r in range(_R):
        node = 0
        for i in range(_DEPTH):
            bit = (r >> (_DEPTH - 1 - i)) & 1
            a[node + _NPAD * bit, r] = 1.0
            node = node * 2 + 1 + bit
    return a

_ROUTE_A = _route_matrix()


def _dtr_kernel(x_ref, w_ref, b_ref, a_ref, out_ref):
    z = jnp.dot(x_ref[...], w_ref[...],
                preferred_element_type=jnp.float32) + b_ref[...]
    u = jnp.log1p(jnp.exp(-jnp.abs(z)))
    sp = jnp.concatenate(
        [u + jnp.maximum(-z, 0.0), u + jnp.maximum(z, 0.0)], axis=1)
    s = jnp.dot(sp, a_ref[...], preferred_element_type=jnp.float32)
    out_ref[...] = jnp.exp(-s)


@functools.partial(jax.jit, static_argnames=("interpret",))
def kernel(x, W, b, interpret=False):
    B, D = x.shape
    n_nodes = W.shape[1]
    tb = 512
    w_pad = jnp.pad(W, ((0, 0), (0, _NPAD - n_nodes)))
    b_pad = jnp.pad(b, (0, _NPAD - n_nodes)).reshape(1, _NPAD)
    a_mat = jnp.asarray(_ROUTE_A)
    return pl.pallas_call(
        _dtr_kernel,
        grid=(B // tb,),
        in_specs=[
            pl.BlockSpec((tb, D), lambda i: (i, 0)),
            pl.BlockSpec((D, _NPAD), lambda i: (0, 0)),
            pl.BlockSpec((1, _NPAD), lambda i: (0, 0)),
            pl.BlockSpec((2 * _NPAD, _R), lambda i: (0, 0)),
        ],
        out_specs=pl.BlockSpec((tb, _R), lambda i: (i, 0)),
        out_shape=jax.ShapeDtypeStruct((B, _R), jnp.float32),
        compiler_params=pltpu.CompilerParams(
            dimension_semantics=("arbitrary",)),
        interpret=interpret,
    )(x, w_pad, b_pad, a_mat)


# speedup vs baseline: 4.3542x; 1.1371x over previous
import functools

import jax
import jax.numpy as jnp
import numpy as np
from jax.experimental import pallas as pl
from jax.experimental.pallas import tpu as pltpu

_DEPTH = 8
_R = 2 ** _DEPTH
_NPAD = _R


def _route_matrix() -> np.ndarray:
    a = np.zeros((2 * _NPAD, _R), dtype=np.float32)
    for r in range(_R):
        node = 0
        for i in range(_DEPTH):
            bit = (r >> (_DEPTH - 1 - i)) & 1
            a[node + _NPAD * bit, r] = 1.0
            node = node * 2 + 1 + bit
    return a

_ROUTE_A = _route_matrix()


def _dtr_kernel(x_ref, w_ref, b_ref, a_ref, out_ref):
    z = jnp.dot(x_ref[...], w_ref[...],
                preferred_element_type=jnp.float32) + b_ref[...]
    u = jnp.log1p(jnp.exp(-jnp.abs(z)))
    sp = jnp.concatenate(
        [u + jnp.maximum(-z, 0.0), u + jnp.maximum(z, 0.0)], axis=1)
    s = jnp.dot(sp, a_ref[...], preferred_element_type=jnp.float32)
    out_ref[...] = jnp.exp(-s)


@functools.partial(jax.jit, static_argnames=("interpret",))
def kernel(x, W, b, interpret=False):
    B, D = x.shape
    n_nodes = W.shape[1]
    tb = 1024
    w_pad = jnp.pad(W, ((0, 0), (0, _NPAD - n_nodes)))
    b_pad = jnp.pad(b, (0, _NPAD - n_nodes)).reshape(1, _NPAD)
    a_mat = jnp.asarray(_ROUTE_A)
    return pl.pallas_call(
        _dtr_kernel,
        grid=(B // tb,),
        in_specs=[
            pl.BlockSpec((tb, D), lambda i: (i, 0)),
            pl.BlockSpec((D, _NPAD), lambda i: (0, 0)),
            pl.BlockSpec((1, _NPAD), lambda i: (0, 0)),
            pl.BlockSpec((2 * _NPAD, _R), lambda i: (0, 0)),
        ],
        out_specs=pl.BlockSpec((tb, _R), lambda i: (i, 0)),
        out_shape=jax.ShapeDtypeStruct((B, _R), jnp.float32),
        compiler_params=pltpu.CompilerParams(
            dimension_semantics=("arbitrary",)),
        interpret=interpret,
    )(x, w_pad, b_pad, a_mat)


# speedup vs baseline: 4.4346x; 1.0185x over previous
import functools

import jax
import jax.numpy as jnp
import numpy as np
from jax.experimental import pallas as pl
from jax.experimental.pallas import tpu as pltpu

_DEPTH = 8
_R = 2 ** _DEPTH
_NPAD = _R


def _route_matrix() -> np.ndarray:
    a = np.zeros((2 * _NPAD, _R), dtype=np.float32)
    for r in range(_R):
        node = 0
        for i in range(_DEPTH):
            bit = (r >> (_DEPTH - 1 - i)) & 1
            a[node + _NPAD * bit, r] = 1.0
            node = node * 2 + 1 + bit
    return a

_ROUTE_A = _route_matrix()


def _dtr_kernel(x_ref, w_ref, b_ref, a_ref, out_ref):
    z = jnp.dot(x_ref[...], w_ref[...],
                preferred_element_type=jnp.float32) + b_ref[...]
    u = jnp.log1p(jnp.exp(-jnp.abs(z)))
    sp = jnp.concatenate(
        [u + jnp.maximum(-z, 0.0), u + jnp.maximum(z, 0.0)], axis=1)
    s = jnp.dot(sp, a_ref[...], preferred_element_type=jnp.float32)
    out_ref[...] = jnp.exp(-s)


@functools.partial(jax.jit, static_argnames=("interpret",))
def kernel(x, W, b, interpret=False):
    B, D = x.shape
    n_nodes = W.shape[1]
    tb = 2048
    w_pad = jnp.pad(W, ((0, 0), (0, _NPAD - n_nodes)))
    b_pad = jnp.pad(b, (0, _NPAD - n_nodes)).reshape(1, _NPAD)
    a_mat = jnp.asarray(_ROUTE_A)
    return pl.pallas_call(
        _dtr_kernel,
        grid=(B // tb,),
        in_specs=[
            pl.BlockSpec((tb, D), lambda i: (i, 0)),
            pl.BlockSpec((D, _NPAD), lambda i: (0, 0)),
            pl.BlockSpec((1, _NPAD), lambda i: (0, 0)),
            pl.BlockSpec((2 * _NPAD, _R), lambda i: (0, 0)),
        ],
        out_specs=pl.BlockSpec((tb, _R), lambda i: (i, 0)),
        out_shape=jax.ShapeDtypeStruct((B, _R), jnp.float32),
        compiler_params=pltpu.CompilerParams(
            dimension_semantics=("arbitrary",)),
        interpret=interpret,
    )(x, w_pad, b_pad, a_mat)
